# initial kernel scaffold (unmeasured)
import jax
import jax.numpy as jnp
from jax import lax
from jax.experimental import pallas as pl
from jax.experimental.pallas import tpu as pltpu

N_DEV = 16
M = 1536
N = 1536
BLK = M // N_DEV
H = N_DEV - 1


def kernel(A, B):
    def body(a_ref, b_ref, out_ref, p_ref, rs_send, rs_recv, my_blk, ag_recv,
             rs_send_sems, rs_recv_sems, ag_send_sems, ag_recv_sems):
        my = lax.axis_index("i")
        right = (my + 1) % N_DEV

        p_ref[...] = jnp.dot(
            a_ref[...].astype(jnp.bfloat16),
            b_ref[...].astype(jnp.bfloat16),
            preferred_element_type=jnp.float32,
        )

        for s in range(H):
            idx_s = (my - s) % N_DEV
            rs_send[s] = p_ref[pl.ds(idx_s * BLK, BLK), :].astype(jnp.bfloat16)
            rdma = pltpu.make_async_remote_copy(
                src_ref=rs_send.at[s],
                dst_ref=rs_recv.at[s],
                send_sem=rs_send_sems.at[s],
                recv_sem=rs_recv_sems.at[s],
                device_id=(right,),
                device_id_type=pl.DeviceIdType.MESH,
            )
            rdma.start()
            rdma.wait()
            idx_r = (my - s - 1) % N_DEV
            p_ref[pl.ds(idx_r * BLK, BLK), :] = (
                p_ref[pl.ds(idx_r * BLK, BLK), :] + rs_recv[s].astype(jnp.float32)
            )

        own = (my + 1) % N_DEV
        out_ref[pl.ds(own * BLK, BLK), :] = p_ref[pl.ds(own * BLK, BLK), :]
        my_blk[...] = p_ref[pl.ds(own * BLK, BLK), :].astype(jnp.bfloat16)

        for s in range(H):
            src = my_blk if s == 0 else ag_recv.at[s - 1]
            rdma = pltpu.make_async_remote_copy(
                src_ref=src,
                dst_ref=ag_recv.at[s],
                send_sem=ag_send_sems.at[s],
                recv_sem=ag_recv_sems.at[s],
                device_id=(right,),
                device_id_type=pl.DeviceIdType.MESH,
            )
            rdma.start()
            rdma.wait()
            idx_r = (my - s) % N_DEV
            out_ref[pl.ds(idx_r * BLK, BLK), :] = ag_recv[s].astype(jnp.float32)

    return pl.pallas_call(
        body,
        out_shape=jax.ShapeDtypeStruct((M, N), jnp.float32),
        in_specs=[
            pl.BlockSpec(memory_space=pltpu.VMEM),
            pl.BlockSpec(memory_space=pltpu.VMEM),
        ],
        out_specs=pl.BlockSpec(memory_space=pltpu.VMEM),
        scratch_shapes=[
            pltpu.VMEM((M, N), jnp.float32),
            pltpu.VMEM((H, BLK, N), jnp.bfloat16),
            pltpu.VMEM((H, BLK, N), jnp.bfloat16),
            pltpu.VMEM((BLK, N), jnp.bfloat16),
            pltpu.VMEM((H, BLK, N), jnp.bfloat16),
            pltpu.SemaphoreType.DMA((H,)),
            pltpu.SemaphoreType.DMA((H,)),
            pltpu.SemaphoreType.DMA((H,)),
            pltpu.SemaphoreType.DMA((H,)),
        ],
        compiler_params=pltpu.CompilerParams(collective_id=0),
    )(A, B)


# baseline (device time: 167500 ns/iter reference)
import jax
import jax.numpy as jnp
from jax import lax
from jax.experimental import pallas as pl
from jax.experimental.pallas import tpu as pltpu

N_DEV = 16
M = 1536
N = 1536
BLK = M // N_DEV
H = N_DEV - 1


def kernel(A, B):
    def body(a_ref, b_ref, out_ref, p_ref, rs_send, rs_recv, my_blk, ag_recv,
             rs_send_sems, rs_recv_sems, ag_send_sems, ag_recv_sems):
        my = lax.axis_index("i")
        right = (my + 1) % N_DEV
        left = (my - 1) % N_DEV

        barrier_sem = pltpu.get_barrier_semaphore()
        for nbr in (left, right):
            pl.semaphore_signal(
                barrier_sem, inc=1,
                device_id=(nbr,), device_id_type=pl.DeviceIdType.MESH,
            )
        pl.semaphore_wait(barrier_sem, 2)

        p_ref[...] = jnp.dot(
            a_ref[...].astype(jnp.bfloat16),
            b_ref[...].astype(jnp.bfloat16),
            preferred_element_type=jnp.float32,
        )

        for s in range(H):
            idx_s = (my - s) % N_DEV
            rs_send[s] = p_ref[pl.ds(idx_s * BLK, BLK), :].astype(jnp.bfloat16)
            rdma = pltpu.make_async_remote_copy(
                src_ref=rs_send.at[s],
                dst_ref=rs_recv.at[s],
                send_sem=rs_send_sems.at[s],
                recv_sem=rs_recv_sems.at[s],
                device_id=(right,),
                device_id_type=pl.DeviceIdType.MESH,
            )
            rdma.start()
            rdma.wait()
            idx_r = (my - s - 1) % N_DEV
            p_ref[pl.ds(idx_r * BLK, BLK), :] = (
                p_ref[pl.ds(idx_r * BLK, BLK), :] + rs_recv[s].astype(jnp.float32)
            )

        own = (my + 1) % N_DEV
        out_ref[pl.ds(own * BLK, BLK), :] = p_ref[pl.ds(own * BLK, BLK), :]
        my_blk[...] = p_ref[pl.ds(own * BLK, BLK), :].astype(jnp.bfloat16)

        for s in range(H):
            src = my_blk if s == 0 else ag_recv.at[s - 1]
            rdma = pltpu.make_async_remote_copy(
                src_ref=src,
                dst_ref=ag_recv.at[s],
                send_sem=ag_send_sems.at[s],
                recv_sem=ag_recv_sems.at[s],
                device_id=(right,),
                device_id_type=pl.DeviceIdType.MESH,
            )
            rdma.start()
            rdma.wait()
            idx_r = (my - s) % N_DEV
            out_ref[pl.ds(idx_r * BLK, BLK), :] = ag_recv[s].astype(jnp.float32)

    return pl.pallas_call(
        body,
        out_shape=jax.ShapeDtypeStruct((M, N), jnp.float32),
        in_specs=[
            pl.BlockSpec(memory_space=pltpu.VMEM),
            pl.BlockSpec(memory_space=pltpu.VMEM),
        ],
        out_specs=pl.BlockSpec(memory_space=pltpu.VMEM),
        scratch_shapes=[
            pltpu.VMEM((M, N), jnp.float32),
            pltpu.VMEM((H, BLK, N), jnp.bfloat16),
            pltpu.VMEM((H, BLK, N), jnp.bfloat16),
            pltpu.VMEM((BLK, N), jnp.bfloat16),
            pltpu.VMEM((H, BLK, N), jnp.bfloat16),
            pltpu.SemaphoreType.DMA((H,)),
            pltpu.SemaphoreType.DMA((H,)),
            pltpu.SemaphoreType.DMA((H,)),
            pltpu.SemaphoreType.DMA((H,)),
        ],
        compiler_params=pltpu.CompilerParams(collective_id=0),
    )(A, B)


# device time: 98388 ns/iter; 1.7024x vs baseline; 1.7024x over previous
import jax
import jax.numpy as jnp
from jax import lax
from jax.experimental import pallas as pl
from jax.experimental.pallas import tpu as pltpu

N_DEV = 16
M = 1536
N = 1536
HN = N // 2
PB = M // 4
ZB = PB // 4
BF = jnp.bfloat16
F32 = jnp.float32


def kernel(A, B):
    def body(a_ref, b_ref, out_ref, p_ref,
             p1Ls, p1Lr, p1Rs, p1Rr,
             p2Ls, p2Lr, p2Rs, p2Rr,
             p3Lr, p3Rr, st3L, st3R,
             p4Lr, p4Rr, st4L, st4R,
             s1Ls, s1Lr, s1Rs, s1Rr,
             s2Ls, s2Lr, s2Rs, s2Rr,
             s3Ls, s3Lr, s3Rs, s3Rr,
             s4Ls, s4Lr, s4Rs, s4Rr):
        my = lax.axis_index("i")
        g = my % 4
        z = my // 4
        zbase = my - g

        pl_next = zbase + (g + 1) % 4
        pl_prev = zbase + (g - 1) % 4
        z_next = ((z + 1) % 4) * 4 + g
        z_prev = ((z - 1) % 4) * 4 + g

        barrier_sem = pltpu.get_barrier_semaphore()
        for nbr in (pl_next, pl_prev, z_next, z_prev):
            pl.semaphore_signal(
                barrier_sem, inc=1,
                device_id=(nbr,), device_id_type=pl.DeviceIdType.MESH,
            )
        pl.semaphore_wait(barrier_sem, 4)

        p_ref[...] = jnp.dot(
            a_ref[...].astype(BF), b_ref[...].astype(BF),
            preferred_element_type=F32,
        )

        pending = []

        def rdma(src, dst, ssem, rsem, dev):
            d = pltpu.make_async_remote_copy(
                src_ref=src, dst_ref=dst, send_sem=ssem, recv_sem=rsem,
                device_id=(dev,), device_id_type=pl.DeviceIdType.MESH,
            )
            d.start()
            pending.append(d)
            return d

        for s in range(3):
            iL = (g - s) % 4
            iR = (g + s) % 4
            p1Ls[s] = p_ref[pl.ds(iL * PB, PB), :HN].astype(BF)
            p1Rs[s] = p_ref[pl.ds(iR * PB, PB), HN:].astype(BF)
            dL = rdma(p1Ls.at[s], p1Lr.at[s], s1Ls.at[s], s1Lr.at[s], pl_next)
            dR = rdma(p1Rs.at[s], p1Rr.at[s], s1Rs.at[s], s1Rr.at[s], pl_prev)
            jL = (g - s - 1) % 4
            jR = (g + s + 1) % 4
            dL.wait_recv()
            p_ref[pl.ds(jL * PB, PB), :HN] = (
                p_ref[pl.ds(jL * PB, PB), :HN] + p1Lr[s].astype(F32))
            dR.wait_recv()
            p_ref[pl.ds(jR * PB, PB), HN:] = (
                p_ref[pl.ds(jR * PB, PB), HN:] + p1Rr[s].astype(F32))

        ownL = (g + 1) % 4
        ownR = (g - 1) % 4
        rowL = ownL * PB
        rowR = ownR * PB

        for s in range(3):
            iL = (z - s) % 4
            iR = (z + s) % 4
            p2Ls[s] = p_ref[pl.ds(rowL + iL * ZB, ZB), :HN].astype(BF)
            p2Rs[s] = p_ref[pl.ds(rowR + iR * ZB, ZB), HN:].astype(BF)
            dL = rdma(p2Ls.at[s], p2Lr.at[s], s2Ls.at[s], s2Lr.at[s], z_next)
            dR = rdma(p2Rs.at[s], p2Rr.at[s], s2Rs.at[s], s2Rr.at[s], z_prev)
            jL = (z - s - 1) % 4
            jR = (z + s + 1) % 4
            dL.wait_recv()
            p_ref[pl.ds(rowL + jL * ZB, ZB), :HN] = (
                p_ref[pl.ds(rowL + jL * ZB, ZB), :HN] + p2Lr[s].astype(F32))
            dR.wait_recv()
            p_ref[pl.ds(rowR + jR * ZB, ZB), HN:] = (
                p_ref[pl.ds(rowR + jR * ZB, ZB), HN:] + p2Rr[s].astype(F32))

        ownLz = (z + 1) % 4
        ownRz = (z - 1) % 4

        out_ref[pl.ds(rowL + ownLz * ZB, ZB), :HN] = (
            p_ref[pl.ds(rowL + ownLz * ZB, ZB), :HN])
        out_ref[pl.ds(rowR + ownRz * ZB, ZB), HN:] = (
            p_ref[pl.ds(rowR + ownRz * ZB, ZB), HN:])
        st3L[...] = p_ref[pl.ds(rowL + ownLz * ZB, ZB), :HN].astype(BF)
        st3R[...] = p_ref[pl.ds(rowR + ownRz * ZB, ZB), HN:].astype(BF)
        st4L[pl.ds(ownLz * ZB, ZB), :] = st3L[...]
        st4R[pl.ds(ownRz * ZB, ZB), :] = st3R[...]

        for s in range(3):
            srcL = st3L if s == 0 else p3Lr.at[s - 1]
            srcR = st3R if s == 0 else p3Rr.at[s - 1]
            dL = rdma(srcL, p3Lr.at[s], s3Ls.at[s], s3Lr.at[s], z_next)
            dR = rdma(srcR, p3Rr.at[s], s3Rs.at[s], s3Rr.at[s], z_prev)
            jL = (z - s) % 4
            jR = (z + s) % 4
            dL.wait_recv()
            st4L[pl.ds(jL * ZB, ZB), :] = p3Lr[s]
            out_ref[pl.ds(rowL + jL * ZB, ZB), :HN] = p3Lr[s].astype(F32)
            dR.wait_recv()
            st4R[pl.ds(jR * ZB, ZB), :] = p3Rr[s]
            out_ref[pl.ds(rowR + jR * ZB, ZB), HN:] = p3Rr[s].astype(F32)

        for s in range(3):
            srcL = st4L if s == 0 else p4Lr.at[s - 1]
            srcR = st4R if s == 0 else p4Rr.at[s - 1]
            dL = rdma(srcL, p4Lr.at[s], s4Ls.at[s], s4Lr.at[s], pl_next)
            dR = rdma(srcR, p4Rr.at[s], s4Rs.at[s], s4Rr.at[s], pl_prev)
            jL = (g - s) % 4
            jR = (g + s) % 4
            dL.wait_recv()
            out_ref[pl.ds(jL * PB, PB), :HN] = p4Lr[s].astype(F32)
            dR.wait_recv()
            out_ref[pl.ds(jR * PB, PB), HN:] = p4Rr[s].astype(F32)

        for d in pending:
            d.wait_send()

    return pl.pallas_call(
        body,
        out_shape=jax.ShapeDtypeStruct((M, N), F32),
        in_specs=[
            pl.BlockSpec(memory_space=pltpu.VMEM),
            pl.BlockSpec(memory_space=pltpu.VMEM),
        ],
        out_specs=pl.BlockSpec(memory_space=pltpu.VMEM),
        scratch_shapes=[
            pltpu.VMEM((M, N), F32),
            pltpu.VMEM((3, PB, HN), BF),
            pltpu.VMEM((3, PB, HN), BF),
            pltpu.VMEM((3, PB, HN), BF),
            pltpu.VMEM((3, PB, HN), BF),
            pltpu.VMEM((3, ZB, HN), BF),
            pltpu.VMEM((3, ZB, HN), BF),
            pltpu.VMEM((3, ZB, HN), BF),
            pltpu.VMEM((3, ZB, HN), BF),
            pltpu.VMEM((3, ZB, HN), BF),
            pltpu.VMEM((3, ZB, HN), BF),
            pltpu.VMEM((ZB, HN), BF),
            pltpu.VMEM((ZB, HN), BF),
            pltpu.VMEM((3, PB, HN), BF),
            pltpu.VMEM((3, PB, HN), BF),
            pltpu.VMEM((PB, HN), BF),
            pltpu.VMEM((PB, HN), BF),
        ] + [pltpu.SemaphoreType.DMA((3,))] * 16,
        compiler_params=pltpu.CompilerParams(collective_id=0),
    )(A, B)


# device time: 88052 ns/iter; 1.9023x vs baseline; 1.1174x over previous
import jax
import jax.numpy as jnp
from jax import lax
from jax.experimental import pallas as pl
from jax.experimental.pallas import tpu as pltpu

N_DEV = 16
M = 1536
N = 1536
HN = N // 2
PB = M // 4
ZB = PB // 4
BF = jnp.bfloat16
F32 = jnp.float32


def kernel(A, B):
    def body(a_ref, b_ref, out_ref, p_ref,
             p1Ls, p1Lr, p1Rs, p1Rr,
             p2Ls, p2Lr, p2Rs, p2Rr,
             p3Lr, p3Rr, st3L, st3R,
             p4Lr, p4Rr, st4L, st4R,
             s1Ls, s1Lr, s1Rs, s1Rr,
             s2Ls, s2Lr, s2Rs, s2Rr,
             s3Ls, s3Lr, s3Rs, s3Rr,
             s4Ls, s4Lr, s4Rs, s4Rr):
        my = lax.axis_index("i")
        g = my % 4
        z = my // 4
        zbase = my - g

        pl_next = zbase + (g + 1) % 4
        pl_prev = zbase + (g - 1) % 4
        z_next = ((z + 1) % 4) * 4 + g
        z_prev = ((z - 1) % 4) * 4 + g

        barrier_sem = pltpu.get_barrier_semaphore()
        for nbr in (pl_next, pl_prev, z_next, z_prev):
            pl.semaphore_signal(
                barrier_sem, inc=1,
                device_id=(nbr,), device_id_type=pl.DeviceIdType.MESH,
            )
        pl.semaphore_wait(barrier_sem, 4)

        pending = []

        def rdma(src, dst, ssem, rsem, dev):
            d = pltpu.make_async_remote_copy(
                src_ref=src, dst_ref=dst, send_sem=ssem, recv_sem=rsem,
                device_id=(dev,), device_id_type=pl.DeviceIdType.MESH,
            )
            d.start()
            pending.append(d)
            return d

        def dot_stripe(i):
            p_ref[pl.ds(i * PB, PB), :] = jnp.dot(
                a_ref[pl.ds(i * PB, PB), :].astype(BF),
                b_ref[...].astype(BF),
                preferred_element_type=F32,
            )

        dot_stripe(g)
        p1Ls[0] = p_ref[pl.ds(g * PB, PB), :HN].astype(BF)
        p1Rs[0] = p_ref[pl.ds(g * PB, PB), HN:].astype(BF)
        d1L = [rdma(p1Ls.at[0], p1Lr.at[0], s1Ls.at[0], s1Lr.at[0], pl_next)]
        d1R = [rdma(p1Rs.at[0], p1Rr.at[0], s1Rs.at[0], s1Rr.at[0], pl_prev)]
        dot_stripe((g + 1) % 4)
        dot_stripe((g + 3) % 4)

        for s in range(2):
            jL = (g - s - 1) % 4
            jR = (g + s + 1) % 4
            d1L[s].wait_recv()
            p1Ls[s + 1] = (
                p_ref[pl.ds(jL * PB, PB), :HN] + p1Lr[s].astype(F32)
            ).astype(BF)
            d1L.append(rdma(p1Ls.at[s + 1], p1Lr.at[s + 1],
                            s1Ls.at[s + 1], s1Lr.at[s + 1], pl_next))
            d1R[s].wait_recv()
            p1Rs[s + 1] = (
                p_ref[pl.ds(jR * PB, PB), HN:] + p1Rr[s].astype(F32)
            ).astype(BF)
            d1R.append(rdma(p1Rs.at[s + 1], p1Rr.at[s + 1],
                            s1Rs.at[s + 1], s1Rr.at[s + 1], pl_prev))
            if s == 0:
                dot_stripe((g + 2) % 4)

        ownL = (g + 1) % 4
        ownR = (g - 1) % 4
        rowL = ownL * PB
        rowR = ownR * PB
        ownLz = (z + 1) % 4
        ownRz = (z - 1) % 4

        d1L[2].wait_recv()
        p2Ls[0] = (
            p_ref[pl.ds(rowL + z * ZB, ZB), :HN]
            + p1Lr[2, pl.ds(z * ZB, ZB), :].astype(F32)
        ).astype(BF)
        d2L = [rdma(p2Ls.at[0], p2Lr.at[0], s2Ls.at[0], s2Lr.at[0], z_next)]
        d1R[2].wait_recv()
        p2Rs[0] = (
            p_ref[pl.ds(rowR + z * ZB, ZB), HN:]
            + p1Rr[2, pl.ds(z * ZB, ZB), :].astype(F32)
        ).astype(BF)
        d2R = [rdma(p2Rs.at[0], p2Rr.at[0], s2Rs.at[0], s2Rr.at[0], z_prev)]
        p_ref[pl.ds(rowL, PB), :HN] = (
            p_ref[pl.ds(rowL, PB), :HN] + p1Lr[2].astype(F32))
        p_ref[pl.ds(rowR, PB), HN:] = (
            p_ref[pl.ds(rowR, PB), HN:] + p1Rr[2].astype(F32))

        for s in range(2):
            jL = (z - s - 1) % 4
            jR = (z + s + 1) % 4
            d2L[s].wait_recv()
            p2Ls[s + 1] = (
                p_ref[pl.ds(rowL + jL * ZB, ZB), :HN] + p2Lr[s].astype(F32)
            ).astype(BF)
            d2L.append(rdma(p2Ls.at[s + 1], p2Lr.at[s + 1],
                            s2Ls.at[s + 1], s2Lr.at[s + 1], z_next))
            d2R[s].wait_recv()
            p2Rs[s + 1] = (
                p_ref[pl.ds(rowR + jR * ZB, ZB), HN:] + p2Rr[s].astype(F32)
            ).astype(BF)
            d2R.append(rdma(p2Rs.at[s + 1], p2Rr.at[s + 1],
                            s2Rs.at[s + 1], s2Rr.at[s + 1], z_prev))

        d2L[2].wait_recv()
        vL = (p_ref[pl.ds(rowL + ownLz * ZB, ZB), :HN]
              + p2Lr[2].astype(F32))
        st3L[...] = vL.astype(BF)
        d3L = [rdma(st3L, p3Lr.at[0], s3Ls.at[0], s3Lr.at[0], z_next)]
        d2R[2].wait_recv()
        vR = (p_ref[pl.ds(rowR + ownRz * ZB, ZB), HN:]
              + p2Rr[2].astype(F32))
        st3R[...] = vR.astype(BF)
        d3R = [rdma(st3R, p3Rr.at[0], s3Rs.at[0], s3Rr.at[0], z_prev)]
        out_ref[pl.ds(rowL + ownLz * ZB, ZB), :HN] = vL
        out_ref[pl.ds(rowR + ownRz * ZB, ZB), HN:] = vR
        st4L[pl.ds(ownLz * ZB, ZB), :] = st3L[...]
        st4R[pl.ds(ownRz * ZB, ZB), :] = st3R[...]

        for s in range(2):
            jL = (z - s) % 4
            jR = (z + s) % 4
            d3L[s].wait_recv()
            d3L.append(rdma(p3Lr.at[s], p3Lr.at[s + 1],
                            s3Ls.at[s + 1], s3Lr.at[s + 1], z_next))
            d3R[s].wait_recv()
            d3R.append(rdma(p3Rr.at[s], p3Rr.at[s + 1],
                            s3Rs.at[s + 1], s3Rr.at[s + 1], z_prev))
            st4L[pl.ds(jL * ZB, ZB), :] = p3Lr[s]
            out_ref[pl.ds(rowL + jL * ZB, ZB), :HN] = p3Lr[s].astype(F32)
            st4R[pl.ds(jR * ZB, ZB), :] = p3Rr[s]
            out_ref[pl.ds(rowR + jR * ZB, ZB), HN:] = p3Rr[s].astype(F32)

        jL = (z - 2) % 4
        jR = (z + 2) % 4
        d3L[2].wait_recv()
        st4L[pl.ds(jL * ZB, ZB), :] = p3Lr[2]
        d3R[2].wait_recv()
        st4R[pl.ds(jR * ZB, ZB), :] = p3Rr[2]
        d4L = [rdma(st4L, p4Lr.at[0], s4Ls.at[0], s4Lr.at[0], pl_next)]
        d4R = [rdma(st4R, p4Rr.at[0], s4Rs.at[0], s4Rr.at[0], pl_prev)]
        out_ref[pl.ds(rowL + jL * ZB, ZB), :HN] = p3Lr[2].astype(F32)
        out_ref[pl.ds(rowR + jR * ZB, ZB), HN:] = p3Rr[2].astype(F32)

        for s in range(2):
            jL = (g - s) % 4
            jR = (g + s) % 4
            d4L[s].wait_recv()
            d4L.append(rdma(p4Lr.at[s], p4Lr.at[s + 1],
                            s4Ls.at[s + 1], s4Lr.at[s + 1], pl_next))
            d4R[s].wait_recv()
            d4R.append(rdma(p4Rr.at[s], p4Rr.at[s + 1],
                            s4Rs.at[s + 1], s4Rr.at[s + 1], pl_prev))
            out_ref[pl.ds(jL * PB, PB), :HN] = p4Lr[s].astype(F32)
            out_ref[pl.ds(jR * PB, PB), HN:] = p4Rr[s].astype(F32)
        d4L[2].wait_recv()
        out_ref[pl.ds(((g - 2) % 4) * PB, PB), :HN] = p4Lr[2].astype(F32)
        d4R[2].wait_recv()
        out_ref[pl.ds(((g + 2) % 4) * PB, PB), HN:] = p4Rr[2].astype(F32)

        for d in pending:
            d.wait_send()

    return pl.pallas_call(
        body,
        out_shape=jax.ShapeDtypeStruct((M, N), F32),
        in_specs=[
            pl.BlockSpec(memory_space=pltpu.VMEM),
            pl.BlockSpec(memory_space=pltpu.VMEM),
        ],
        out_specs=pl.BlockSpec(memory_space=pltpu.VMEM),
        scratch_shapes=[
            pltpu.VMEM((M, N), F32),
            pltpu.VMEM((3, PB, HN), BF),
            pltpu.VMEM((3, PB, HN), BF),
            pltpu.VMEM((3, PB, HN), BF),
            pltpu.VMEM((3, PB, HN), BF),
            pltpu.VMEM((3, ZB, HN), BF),
            pltpu.VMEM((3, ZB, HN), BF),
            pltpu.VMEM((3, ZB, HN), BF),
            pltpu.VMEM((3, ZB, HN), BF),
            pltpu.VMEM((3, ZB, HN), BF),
            pltpu.VMEM((3, ZB, HN), BF),
            pltpu.VMEM((ZB, HN), BF),
            pltpu.VMEM((ZB, HN), BF),
            pltpu.VMEM((3, PB, HN), BF),
            pltpu.VMEM((3, PB, HN), BF),
            pltpu.VMEM((PB, HN), BF),
            pltpu.VMEM((PB, HN), BF),
        ] + [pltpu.SemaphoreType.DMA((3,))] * 16,
        compiler_params=pltpu.CompilerParams(collective_id=0),
    )(A, B)


# device time: 75576 ns/iter; 2.2163x vs baseline; 1.1651x over previous
import jax
import jax.numpy as jnp
from jax import lax
from jax.experimental import pallas as pl
from jax.experimental.pallas import tpu as pltpu

N_DEV = 16
M = 1536
N = 1536
HN = N // 2
PB = M // 4
ZB = PB // 4
BF = jnp.bfloat16
F32 = jnp.float32


def kernel(A, B):
    def body(a_ref, b_ref, out_ref, p_ref,
             p1Ls, p1Lr, p1Rs, p1Rr,
             p2Ls, p2Lr, p2Rs, p2Rr,
             p3Lr, p3Rr, st3L, st3R,
             p4Lr, p4Rr,
             s1Ls, s1Lr, s1Rs, s1Rr,
             s2Ls, s2Lr, s2Rs, s2Rr,
             s3Ls, s3Lr, s3Rs, s3Rr,
             s4Ls, s4Lr, s4Rs, s4Rr):
        my = lax.axis_index("i")
        g = my % 4
        z = my // 4
        zbase = my - g

        pl_next = zbase + (g + 1) % 4
        pl_prev = zbase + (g - 1) % 4
        z_next = ((z + 1) % 4) * 4 + g
        z_prev = ((z - 1) % 4) * 4 + g

        barrier_sem = pltpu.get_barrier_semaphore()
        for nbr in (pl_next, pl_prev, z_next, z_prev):
            pl.semaphore_signal(
                barrier_sem, inc=1,
                device_id=(nbr,), device_id_type=pl.DeviceIdType.MESH,
            )
        pl.semaphore_wait(barrier_sem, 4)

        pending = []

        def rdma(src, dst, ssem, rsem, dev):
            d = pltpu.make_async_remote_copy(
                src_ref=src, dst_ref=dst, send_sem=ssem, recv_sem=rsem,
                device_id=(dev,), device_id_type=pl.DeviceIdType.MESH,
            )
            d.start()
            pending.append(d)
            return d

        def dot_stripe(i):
            p_ref[pl.ds(i * PB, PB), :] = jnp.dot(
                a_ref[pl.ds(i * PB, PB), :].astype(BF),
                b_ref[...].astype(BF),
                preferred_element_type=F32,
            )

        dot_stripe(g)
        p1Ls[0] = p_ref[pl.ds(g * PB, PB), :HN].astype(BF)
        p1Rs[0] = p_ref[pl.ds(g * PB, PB), HN:].astype(BF)
        d1L = [rdma(p1Ls.at[0], p1Lr.at[0], s1Ls.at[0], s1Lr.at[0], pl_next)]
        d1R = [rdma(p1Rs.at[0], p1Rr.at[0], s1Rs.at[0], s1Rr.at[0], pl_prev)]
        dot_stripe((g + 1) % 4)
        dot_stripe((g + 3) % 4)

        d1L[0].wait_recv()
        p1Ls[1] = (
            p_ref[pl.ds(((g - 1) % 4) * PB, PB), :HN] + p1Lr[0].astype(F32)
        ).astype(BF)
        d1L.append(rdma(p1Ls.at[1], p1Lr.at[1], s1Ls.at[1], s1Lr.at[1],
                        pl_next))
        d1R[0].wait_recv()
        p1Rs[1] = (
            p_ref[pl.ds(((g + 1) % 4) * PB, PB), HN:] + p1Rr[0].astype(F32)
        ).astype(BF)
        d1R.append(rdma(p1Rs.at[1], p1Rr.at[1], s1Rs.at[1], s1Rr.at[1],
                        pl_prev))
        dot_stripe((g + 2) % 4)

        jL2 = (g - 2) % 4
        jR2 = (g + 2) % 4
        d1L[1].wait_recv()
        d1Lz = []
        for k in range(4):
            zk = ((z - k) % 4) * ZB
            p1Ls[2, pl.ds(k * ZB, ZB), :] = (
                p_ref[pl.ds(jL2 * PB + zk, ZB), :HN]
                + p1Lr[1, pl.ds(zk, ZB), :].astype(F32)
            ).astype(BF)
            d1Lz.append(rdma(p1Ls.at[2, pl.ds(k * ZB, ZB)],
                             p1Lr.at[2, pl.ds(k * ZB, ZB)],
                             s1Ls.at[2 + k], s1Lr.at[2 + k], pl_next))
        d1R[1].wait_recv()
        d1Rz = []
        for k in range(4):
            zk = ((z + k) % 4) * ZB
            p1Rs[2, pl.ds(k * ZB, ZB), :] = (
                p_ref[pl.ds(jR2 * PB + zk, ZB), HN:]
                + p1Rr[1, pl.ds(zk, ZB), :].astype(F32)
            ).astype(BF)
            d1Rz.append(rdma(p1Rs.at[2, pl.ds(k * ZB, ZB)],
                             p1Rr.at[2, pl.ds(k * ZB, ZB)],
                             s1Rs.at[2 + k], s1Rr.at[2 + k], pl_prev))

        ownL = (g + 1) % 4
        ownR = (g - 1) % 4
        rowL = ownL * PB
        rowR = ownR * PB
        ownLz = (z + 1) % 4
        ownRz = (z - 1) % 4

        d1Lz[0].wait_recv()
        p2Ls[0] = (
            p_ref[pl.ds(rowL + z * ZB, ZB), :HN]
            + p1Lr[2, pl.ds(0, ZB), :].astype(F32)
        ).astype(BF)
        d2L = [rdma(p2Ls.at[0], p2Lr.at[0], s2Ls.at[0], s2Lr.at[0], z_next)]
        d1Rz[0].wait_recv()
        p2Rs[0] = (
            p_ref[pl.ds(rowR + z * ZB, ZB), HN:]
            + p1Rr[2, pl.ds(0, ZB), :].astype(F32)
        ).astype(BF)
        d2R = [rdma(p2Rs.at[0], p2Rr.at[0], s2Rs.at[0], s2Rr.at[0], z_prev)]

        for s in range(2):
            jL = (z - s - 1) % 4
            jR = (z + s + 1) % 4
            d2L[s].wait_recv()
            d1Lz[s + 1].wait_recv()
            p2Ls[s + 1] = (
                p_ref[pl.ds(rowL + jL * ZB, ZB), :HN]
                + p1Lr[2, pl.ds((s + 1) * ZB, ZB), :].astype(F32)
                + p2Lr[s].astype(F32)
            ).astype(BF)
            d2L.append(rdma(p2Ls.at[s + 1], p2Lr.at[s + 1],
                            s2Ls.at[s + 1], s2Lr.at[s + 1], z_next))
            d2R[s].wait_recv()
            d1Rz[s + 1].wait_recv()
            p2Rs[s + 1] = (
                p_ref[pl.ds(rowR + jR * ZB, ZB), HN:]
                + p1Rr[2, pl.ds((s + 1) * ZB, ZB), :].astype(F32)
                + p2Rr[s].astype(F32)
            ).astype(BF)
            d2R.append(rdma(p2Rs.at[s + 1], p2Rr.at[s + 1],
                            s2Rs.at[s + 1], s2Rr.at[s + 1], z_prev))

        d2L[2].wait_recv()
        d1Lz[3].wait_recv()
        vL = (p_ref[pl.ds(rowL + ownLz * ZB, ZB), :HN]
              + p1Lr[2, pl.ds(3 * ZB, ZB), :].astype(F32)
              + p2Lr[2].astype(F32))
        st3L[...] = vL.astype(BF)
        d3L = [rdma(st3L, p3Lr.at[0], s3Ls.at[0], s3Lr.at[0], z_next)]
        d4L = [rdma(st3L, p4Lr.at[0, pl.ds(0, ZB)],
                    s4Ls.at[0], s4Lr.at[0], pl_next)]
        d2R[2].wait_recv()
        d1Rz[3].wait_recv()
        vR = (p_ref[pl.ds(rowR + ownRz * ZB, ZB), HN:]
              + p1Rr[2, pl.ds(3 * ZB, ZB), :].astype(F32)
              + p2Rr[2].astype(F32))
        st3R[...] = vR.astype(BF)
        d3R = [rdma(st3R, p3Rr.at[0], s3Rs.at[0], s3Rr.at[0], z_prev)]
        d4R = [rdma(st3R, p4Rr.at[0, pl.ds(0, ZB)],
                    s4Rs.at[0], s4Rr.at[0], pl_prev)]
        out_ref[pl.ds(rowL + ownLz * ZB, ZB), :HN] = vL
        out_ref[pl.ds(rowR + ownRz * ZB, ZB), HN:] = vR

        for j in range(3):
            jL = (z - j) % 4
            jR = (z + j) % 4
            d3L[j].wait_recv()
            if j < 2:
                d3L.append(rdma(p3Lr.at[j], p3Lr.at[j + 1],
                                s3Ls.at[j + 1], s3Lr.at[j + 1], z_next))
            d4L.append(rdma(p3Lr.at[j], p4Lr.at[0, pl.ds((j + 1) * ZB, ZB)],
                            s4Ls.at[j + 1], s4Lr.at[j + 1], pl_next))
            d3R[j].wait_recv()
            if j < 2:
                d3R.append(rdma(p3Rr.at[j], p3Rr.at[j + 1],
                                s3Rs.at[j + 1], s3Rr.at[j + 1], z_prev))
            d4R.append(rdma(p3Rr.at[j], p4Rr.at[0, pl.ds((j + 1) * ZB, ZB)],
                            s4Rs.at[j + 1], s4Rr.at[j + 1], pl_prev))
            out_ref[pl.ds(rowL + jL * ZB, ZB), :HN] = p3Lr[j].astype(F32)
            out_ref[pl.ds(rowR + jR * ZB, ZB), HN:] = p3Rr[j].astype(F32)

        for s in range(3):
            jL = (g - s) % 4
            jR = (g + s) % 4
            for k in range(4):
                i = s * 4 + k
                d4L[i].wait_recv()
                if s < 2:
                    d4L.append(rdma(p4Lr.at[s, pl.ds(k * ZB, ZB)],
                                    p4Lr.at[s + 1, pl.ds(k * ZB, ZB)],
                                    s4Ls.at[i + 4], s4Lr.at[i + 4], pl_next))
                d4R[i].wait_recv()
                if s < 2:
                    d4R.append(rdma(p4Rr.at[s, pl.ds(k * ZB, ZB)],
                                    p4Rr.at[s + 1, pl.ds(k * ZB, ZB)],
                                    s4Rs.at[i + 4], s4Rr.at[i + 4], pl_prev))
                out_ref[pl.ds(jL * PB + ((z + 1 - k) % 4) * ZB, ZB), :HN] = (
                    p4Lr[s, pl.ds(k * ZB, ZB), :].astype(F32))
                out_ref[pl.ds(jR * PB + ((z - 1 + k) % 4) * ZB, ZB), HN:] = (
                    p4Rr[s, pl.ds(k * ZB, ZB), :].astype(F32))

        for d in pending:
            d.wait_send()

    return pl.pallas_call(
        body,
        out_shape=jax.ShapeDtypeStruct((M, N), F32),
        in_specs=[
            pl.BlockSpec(memory_space=pltpu.VMEM),
            pl.BlockSpec(memory_space=pltpu.VMEM),
        ],
        out_specs=pl.BlockSpec(memory_space=pltpu.VMEM),
        scratch_shapes=[
            pltpu.VMEM((M, N), F32),
            pltpu.VMEM((3, PB, HN), BF),
            pltpu.VMEM((3, PB, HN), BF),
            pltpu.VMEM((3, PB, HN), BF),
            pltpu.VMEM((3, PB, HN), BF),
            pltpu.VMEM((3, ZB, HN), BF),
            pltpu.VMEM((3, ZB, HN), BF),
            pltpu.VMEM((3, ZB, HN), BF),
            pltpu.VMEM((3, ZB, HN), BF),
            pltpu.VMEM((3, ZB, HN), BF),
            pltpu.VMEM((3, ZB, HN), BF),
            pltpu.VMEM((ZB, HN), BF),
            pltpu.VMEM((ZB, HN), BF),
            pltpu.VMEM((3, PB, HN), BF),
            pltpu.VMEM((3, PB, HN), BF),
            pltpu.SemaphoreType.DMA((6,)),
            pltpu.SemaphoreType.DMA((6,)),
            pltpu.SemaphoreType.DMA((6,)),
            pltpu.SemaphoreType.DMA((6,)),
            pltpu.SemaphoreType.DMA((3,)),
            pltpu.SemaphoreType.DMA((3,)),
            pltpu.SemaphoreType.DMA((3,)),
            pltpu.SemaphoreType.DMA((3,)),
            pltpu.SemaphoreType.DMA((3,)),
            pltpu.SemaphoreType.DMA((3,)),
            pltpu.SemaphoreType.DMA((3,)),
            pltpu.SemaphoreType.DMA((3,)),
            pltpu.SemaphoreType.DMA((12,)),
            pltpu.SemaphoreType.DMA((12,)),
            pltpu.SemaphoreType.DMA((12,)),
            pltpu.SemaphoreType.DMA((12,)),
        ],
        compiler_params=pltpu.CompilerParams(collective_id=0),
    )(A, B)


# device time: 74647 ns/iter; 2.2439x vs baseline; 1.0124x over previous
import jax
import jax.numpy as jnp
from jax import lax
from jax.experimental import pallas as pl
from jax.experimental.pallas import tpu as pltpu

N_DEV = 16
M = 1536
N = 1536
HN = N // 2
PB = M // 4
ZB = PB // 4
BF = jnp.bfloat16
F32 = jnp.float32


def kernel(A, B):
    def body(a_ref, b_ref, out_ref, p_ref,
             p1Ls, p1Lr, p1Rs, p1Rr,
             p2Ls, p2Lr, p2Rs, p2Rr,
             p3Lr, p3Rr, st3L, st3R,
             p4Lr, p4Rr,
             s1Ls, s1Lr, s1Rs, s1Rr,
             s2Ls, s2Lr, s2Rs, s2Rr,
             s3Ls, s3Lr, s3Rs, s3Rr,
             s4Ls, s4Lr, s4Rs, s4Rr):
        my = lax.axis_index("i")
        g = my % 4
        z = my // 4
        zbase = my - g

        pl_next = zbase + (g + 1) % 4
        pl_prev = zbase + (g - 1) % 4
        z_next = ((z + 1) % 4) * 4 + g
        z_prev = ((z - 1) % 4) * 4 + g

        barrier_sem = pltpu.get_barrier_semaphore()
        for nbr in (pl_next, pl_prev, z_next, z_prev):
            pl.semaphore_signal(
                barrier_sem, inc=1,
                device_id=(nbr,), device_id_type=pl.DeviceIdType.MESH,
            )
        pl.semaphore_wait(barrier_sem, 4)

        pending = []

        def rdma(src, dst, ssem, rsem, dev):
            d = pltpu.make_async_remote_copy(
                src_ref=src, dst_ref=dst, send_sem=ssem, recv_sem=rsem,
                device_id=(dev,), device_id_type=pl.DeviceIdType.MESH,
            )
            d.start()
            pending.append(d)
            return d

        def dot_half(i, left):
            cols = pl.ds(0, HN) if left else pl.ds(HN, HN)
            p_ref[pl.ds(i * PB, PB), cols] = jnp.dot(
                a_ref[pl.ds(i * PB, PB), :].astype(BF),
                b_ref[:, cols].astype(BF),
                preferred_element_type=F32,
            )

        zL = [((z - k) % 4) * ZB for k in range(4)]
        zR = [((z + k) % 4) * ZB for k in range(4)]

        dot_half(g, True)
        dot_half(g, False)
        d1L, d1R = [], []
        for k in range(4):
            p1Ls[0, pl.ds(k * ZB, ZB), :] = (
                p_ref[pl.ds(g * PB + zL[k], ZB), :HN].astype(BF))
            d1L.append(rdma(p1Ls.at[0, pl.ds(k * ZB, ZB)],
                            p1Lr.at[0, pl.ds(k * ZB, ZB)],
                            s1Ls.at[k], s1Lr.at[k], pl_next))
        for k in range(4):
            p1Rs[0, pl.ds(k * ZB, ZB), :] = (
                p_ref[pl.ds(g * PB + zR[k], ZB), HN:].astype(BF))
            d1R.append(rdma(p1Rs.at[0, pl.ds(k * ZB, ZB)],
                            p1Rr.at[0, pl.ds(k * ZB, ZB)],
                            s1Rs.at[k], s1Rr.at[k], pl_prev))

        dot_half((g + 3) % 4, True)
        jL1 = (g - 1) % 4
        for k in range(4):
            d1L[k].wait_recv()
            p1Ls[1, pl.ds(k * ZB, ZB), :] = (
                p_ref[pl.ds(jL1 * PB + zL[k], ZB), :HN]
                + p1Lr[0, pl.ds(k * ZB, ZB), :].astype(F32)
            ).astype(BF)
            d1L.append(rdma(p1Ls.at[1, pl.ds(k * ZB, ZB)],
                            p1Lr.at[1, pl.ds(k * ZB, ZB)],
                            s1Ls.at[4 + k], s1Lr.at[4 + k], pl_next))

        dot_half((g + 1) % 4, False)
        jR1 = (g + 1) % 4
        for k in range(4):
            d1R[k].wait_recv()
            p1Rs[1, pl.ds(k * ZB, ZB), :] = (
                p_ref[pl.ds(jR1 * PB + zR[k], ZB), HN:]
                + p1Rr[0, pl.ds(k * ZB, ZB), :].astype(F32)
            ).astype(BF)
            d1R.append(rdma(p1Rs.at[1, pl.ds(k * ZB, ZB)],
                            p1Rr.at[1, pl.ds(k * ZB, ZB)],
                            s1Rs.at[4 + k], s1Rr.at[4 + k], pl_prev))

        dot_half((g + 2) % 4, True)
        jL2 = (g - 2) % 4
        d1Lz = []
        for k in range(4):
            d1L[4 + k].wait_recv()
            p1Ls[2, pl.ds(k * ZB, ZB), :] = (
                p_ref[pl.ds(jL2 * PB + zL[k], ZB), :HN]
                + p1Lr[1, pl.ds(k * ZB, ZB), :].astype(F32)
            ).astype(BF)
            d1Lz.append(rdma(p1Ls.at[2, pl.ds(k * ZB, ZB)],
                             p1Lr.at[2, pl.ds(k * ZB, ZB)],
                             s1Ls.at[8 + k], s1Lr.at[8 + k], pl_next))

        dot_half((g + 2) % 4, False)
        jR2 = (g + 2) % 4
        d1Rz = []
        for k in range(4):
            d1R[4 + k].wait_recv()
            p1Rs[2, pl.ds(k * ZB, ZB), :] = (
                p_ref[pl.ds(jR2 * PB + zR[k], ZB), HN:]
                + p1Rr[1, pl.ds(k * ZB, ZB), :].astype(F32)
            ).astype(BF)
            d1Rz.append(rdma(p1Rs.at[2, pl.ds(k * ZB, ZB)],
                             p1Rr.at[2, pl.ds(k * ZB, ZB)],
                             s1Rs.at[8 + k], s1Rr.at[8 + k], pl_prev))

        dot_half((g + 1) % 4, True)
        dot_half((g + 3) % 4, False)

        ownL = (g + 1) % 4
        ownR = (g - 1) % 4
        rowL = ownL * PB
        rowR = ownR * PB
        ownLz = (z + 1) % 4
        ownRz = (z - 1) % 4

        d1Lz[0].wait_recv()
        p2Ls[0] = (
            p_ref[pl.ds(rowL + z * ZB, ZB), :HN]
            + p1Lr[2, pl.ds(0, ZB), :].astype(F32)
        ).astype(BF)
        d2L = [rdma(p2Ls.at[0], p2Lr.at[0], s2Ls.at[0], s2Lr.at[0], z_next)]
        d1Rz[0].wait_recv()
        p2Rs[0] = (
            p_ref[pl.ds(rowR + z * ZB, ZB), HN:]
            + p1Rr[2, pl.ds(0, ZB), :].astype(F32)
        ).astype(BF)
        d2R = [rdma(p2Rs.at[0], p2Rr.at[0], s2Rs.at[0], s2Rr.at[0], z_prev)]

        for s in range(2):
            jL = (z - s - 1) % 4
            jR = (z + s + 1) % 4
            d2L[s].wait_recv()
            d1Lz[s + 1].wait_recv()
            p2Ls[s + 1] = (
                p_ref[pl.ds(rowL + jL * ZB, ZB), :HN]
                + p1Lr[2, pl.ds((s + 1) * ZB, ZB), :].astype(F32)
                + p2Lr[s].astype(F32)
            ).astype(BF)
            d2L.append(rdma(p2Ls.at[s + 1], p2Lr.at[s + 1],
                            s2Ls.at[s + 1], s2Lr.at[s + 1], z_next))
            d2R[s].wait_recv()
            d1Rz[s + 1].wait_recv()
            p2Rs[s + 1] = (
                p_ref[pl.ds(rowR + jR * ZB, ZB), HN:]
                + p1Rr[2, pl.ds((s + 1) * ZB, ZB), :].astype(F32)
                + p2Rr[s].astype(F32)
            ).astype(BF)
            d2R.append(rdma(p2Rs.at[s + 1], p2Rr.at[s + 1],
                            s2Rs.at[s + 1], s2Rr.at[s + 1], z_prev))

        d2L[2].wait_recv()
        d1Lz[3].wait_recv()
        vL = (p_ref[pl.ds(rowL + ownLz * ZB, ZB), :HN]
              + p1Lr[2, pl.ds(3 * ZB, ZB), :].astype(F32)
              + p2Lr[2].astype(F32))
        st3L[...] = vL.astype(BF)
        d3L = [rdma(st3L, p3Lr.at[0], s3Ls.at[0], s3Lr.at[0], z_next)]
        d4L = [rdma(st3L, p4Lr.at[0, pl.ds(0, ZB)],
                    s4Ls.at[0], s4Lr.at[0], pl_next)]
        d2R[2].wait_recv()
        d1Rz[3].wait_recv()
        vR = (p_ref[pl.ds(rowR + ownRz * ZB, ZB), HN:]
              + p1Rr[2, pl.ds(3 * ZB, ZB), :].astype(F32)
              + p2Rr[2].astype(F32))
        st3R[...] = vR.astype(BF)
        d3R = [rdma(st3R, p3Rr.at[0], s3Rs.at[0], s3Rr.at[0], z_prev)]
        d4R = [rdma(st3R, p4Rr.at[0, pl.ds(0, ZB)],
                    s4Rs.at[0], s4Rr.at[0], pl_prev)]
        out_ref[pl.ds(rowL + ownLz * ZB, ZB), :HN] = vL
        out_ref[pl.ds(rowR + ownRz * ZB, ZB), HN:] = vR

        for j in range(3):
            jL = (z - j) % 4
            jR = (z + j) % 4
            d3L[j].wait_recv()
            if j < 2:
                d3L.append(rdma(p3Lr.at[j], p3Lr.at[j + 1],
                                s3Ls.at[j + 1], s3Lr.at[j + 1], z_next))
            d4L.append(rdma(p3Lr.at[j], p4Lr.at[0, pl.ds((j + 1) * ZB, ZB)],
                            s4Ls.at[j + 1], s4Lr.at[j + 1], pl_next))
            d3R[j].wait_recv()
            if j < 2:
                d3R.append(rdma(p3Rr.at[j], p3Rr.at[j + 1],
                                s3Rs.at[j + 1], s3Rr.at[j + 1], z_prev))
            d4R.append(rdma(p3Rr.at[j], p4Rr.at[0, pl.ds((j + 1) * ZB, ZB)],
                            s4Rs.at[j + 1], s4Rr.at[j + 1], pl_prev))
            out_ref[pl.ds(rowL + jL * ZB, ZB), :HN] = p3Lr[j].astype(F32)
            out_ref[pl.ds(rowR + jR * ZB, ZB), HN:] = p3Rr[j].astype(F32)

        for s in range(3):
            jL = (g - s) % 4
            jR = (g + s) % 4
            for k in range(4):
                i = s * 4 + k
                d4L[i].wait_recv()
                if s < 2:
                    d4L.append(rdma(p4Lr.at[s, pl.ds(k * ZB, ZB)],
                                    p4Lr.at[s + 1, pl.ds(k * ZB, ZB)],
                                    s4Ls.at[i + 4], s4Lr.at[i + 4], pl_next))
                d4R[i].wait_recv()
                if s < 2:
                    d4R.append(rdma(p4Rr.at[s, pl.ds(k * ZB, ZB)],
                                    p4Rr.at[s + 1, pl.ds(k * ZB, ZB)],
                                    s4Rs.at[i + 4], s4Rr.at[i + 4], pl_prev))
                out_ref[pl.ds(jL * PB + ((z + 1 - k) % 4) * ZB, ZB), :HN] = (
                    p4Lr[s, pl.ds(k * ZB, ZB), :].astype(F32))
                out_ref[pl.ds(jR * PB + ((z - 1 + k) % 4) * ZB, ZB), HN:] = (
                    p4Rr[s, pl.ds(k * ZB, ZB), :].astype(F32))

        for d in pending:
            d.wait_send()

    return pl.pallas_call(
        body,
        out_shape=jax.ShapeDtypeStruct((M, N), F32),
        in_specs=[
            pl.BlockSpec(memory_space=pltpu.VMEM),
            pl.BlockSpec(memory_space=pltpu.VMEM),
        ],
        out_specs=pl.BlockSpec(memory_space=pltpu.VMEM),
        scratch_shapes=[
            pltpu.VMEM((M, N), F32),
            pltpu.VMEM((3, PB, HN), BF),
            pltpu.VMEM((3, PB, HN), BF),
            pltpu.VMEM((3, PB, HN), BF),
            pltpu.VMEM((3, PB, HN), BF),
            pltpu.VMEM((3, ZB, HN), BF),
            pltpu.VMEM((3, ZB, HN), BF),
            pltpu.VMEM((3, ZB, HN), BF),
            pltpu.VMEM((3, ZB, HN), BF),
            pltpu.VMEM((3, ZB, HN), BF),
            pltpu.VMEM((3, ZB, HN), BF),
            pltpu.VMEM((ZB, HN), BF),
            pltpu.VMEM((ZB, HN), BF),
            pltpu.VMEM((3, PB, HN), BF),
            pltpu.VMEM((3, PB, HN), BF),
            pltpu.SemaphoreType.DMA((12,)),
            pltpu.SemaphoreType.DMA((12,)),
            pltpu.SemaphoreType.DMA((12,)),
            pltpu.SemaphoreType.DMA((12,)),
            pltpu.SemaphoreType.DMA((3,)),
            pltpu.SemaphoreType.DMA((3,)),
            pltpu.SemaphoreType.DMA((3,)),
            pltpu.SemaphoreType.DMA((3,)),
            pltpu.SemaphoreType.DMA((3,)),
            pltpu.SemaphoreType.DMA((3,)),
            pltpu.SemaphoreType.DMA((3,)),
            pltpu.SemaphoreType.DMA((3,)),
            pltpu.SemaphoreType.DMA((12,)),
            pltpu.SemaphoreType.DMA((12,)),
            pltpu.SemaphoreType.DMA((12,)),
            pltpu.SemaphoreType.DMA((12,)),
        ],
        compiler_params=pltpu.CompilerParams(collective_id=0),
    )(A, B)


# device time: 71497 ns/iter; 2.3428x vs baseline; 1.0441x over previous
import jax
import jax.numpy as jnp
from jax import lax
from jax.experimental import pallas as pl
from jax.experimental.pallas import tpu as pltpu

N_DEV = 16
M = 1536
N = 1536
HN = N // 2
PB = M // 4
ZB = PB // 4
BF = jnp.bfloat16
F32 = jnp.float32


def kernel(A, B):
    def body(a_ref, b_ref, out_ref, p_ref,
             p1Ls, p1Lr, p1Rs, p1Rr,
             p2Ls, p2Lr, p2Rs, p2Rr,
             s1Ls, s1Lr, s1Rs, s1Rr,
             s2Ls, s2Lr, s2Rs, s2Rr,
             s3Ls, s3Lr, s3Rs, s3Rr,
             s4Ls, s4Lr, s4Rs, s4Rr):
        my = lax.axis_index("i")
        g = my % 4
        z = my // 4
        zbase = my - g

        pl_next = zbase + (g + 1) % 4
        pl_prev = zbase + (g - 1) % 4
        z_next = ((z + 1) % 4) * 4 + g
        z_prev = ((z - 1) % 4) * 4 + g

        barrier_sem = pltpu.get_barrier_semaphore()
        for nbr in (pl_next, pl_prev, z_next, z_prev):
            pl.semaphore_signal(
                barrier_sem, inc=1,
                device_id=(nbr,), device_id_type=pl.DeviceIdType.MESH,
            )
        pl.semaphore_wait(barrier_sem, 4)

        pending = []

        def rdma(src, dst, ssem, rsem, dev):
            d = pltpu.make_async_remote_copy(
                src_ref=src, dst_ref=dst, send_sem=ssem, recv_sem=rsem,
                device_id=(dev,), device_id_type=pl.DeviceIdType.MESH,
            )
            d.start()
            pending.append(d)
            return d

        def dot_half(i, left):
            cols = pl.ds(0, HN) if left else pl.ds(HN, HN)
            p_ref[pl.ds(i * PB, PB), cols] = jnp.dot(
                a_ref[pl.ds(i * PB, PB), :].astype(BF),
                b_ref[:, cols].astype(BF),
                preferred_element_type=F32,
            )

        zL = [((z - k) % 4) * ZB for k in range(4)]
        zR = [((z + k) % 4) * ZB for k in range(4)]

        dot_half(g, True)
        dot_half(g, False)
        d1L, d1R = [], []
        for k in range(4):
            p1Ls[0, pl.ds(k * ZB, ZB), :] = (
                p_ref[pl.ds(g * PB + zL[k], ZB), :HN].astype(BF))
            d1L.append(rdma(p1Ls.at[0, pl.ds(k * ZB, ZB)],
                            p1Lr.at[0, pl.ds(k * ZB, ZB)],
                            s1Ls.at[k], s1Lr.at[k], pl_next))
        for k in range(4):
            p1Rs[0, pl.ds(k * ZB, ZB), :] = (
                p_ref[pl.ds(g * PB + zR[k], ZB), HN:].astype(BF))
            d1R.append(rdma(p1Rs.at[0, pl.ds(k * ZB, ZB)],
                            p1Rr.at[0, pl.ds(k * ZB, ZB)],
                            s1Rs.at[k], s1Rr.at[k], pl_prev))

        dot_half((g + 3) % 4, True)
        jL1 = (g - 1) % 4
        for k in range(4):
            d1L[k].wait_recv()
            p1Ls[1, pl.ds(k * ZB, ZB), :] = (
                p_ref[pl.ds(jL1 * PB + zL[k], ZB), :HN]
                + p1Lr[0, pl.ds(k * ZB, ZB), :].astype(F32)
            ).astype(BF)
            d1L.append(rdma(p1Ls.at[1, pl.ds(k * ZB, ZB)],
                            p1Lr.at[1, pl.ds(k * ZB, ZB)],
                            s1Ls.at[4 + k], s1Lr.at[4 + k], pl_next))

        dot_half((g + 1) % 4, False)
        jR1 = (g + 1) % 4
        for k in range(4):
            d1R[k].wait_recv()
            p1Rs[1, pl.ds(k * ZB, ZB), :] = (
                p_ref[pl.ds(jR1 * PB + zR[k], ZB), HN:]
                + p1Rr[0, pl.ds(k * ZB, ZB), :].astype(F32)
            ).astype(BF)
            d1R.append(rdma(p1Rs.at[1, pl.ds(k * ZB, ZB)],
                            p1Rr.at[1, pl.ds(k * ZB, ZB)],
                            s1Rs.at[4 + k], s1Rr.at[4 + k], pl_prev))

        dot_half((g + 2) % 4, True)
        jL2 = (g - 2) % 4
        d1Lz = []
        for k in range(4):
            d1L[4 + k].wait_recv()
            p1Ls[2, pl.ds(k * ZB, ZB), :] = (
                p_ref[pl.ds(jL2 * PB + zL[k], ZB), :HN]
                + p1Lr[1, pl.ds(k * ZB, ZB), :].astype(F32)
            ).astype(BF)
            d1Lz.append(rdma(p1Ls.at[2, pl.ds(k * ZB, ZB)],
                             p1Lr.at[2, pl.ds(k * ZB, ZB)],
                             s1Ls.at[8 + k], s1Lr.at[8 + k], pl_next))

        dot_half((g + 2) % 4, False)
        jR2 = (g + 2) % 4
        d1Rz = []
        for k in range(4):
            d1R[4 + k].wait_recv()
            p1Rs[2, pl.ds(k * ZB, ZB), :] = (
                p_ref[pl.ds(jR2 * PB + zR[k], ZB), HN:]
                + p1Rr[1, pl.ds(k * ZB, ZB), :].astype(F32)
            ).astype(BF)
            d1Rz.append(rdma(p1Rs.at[2, pl.ds(k * ZB, ZB)],
                             p1Rr.at[2, pl.ds(k * ZB, ZB)],
                             s1Rs.at[8 + k], s1Rr.at[8 + k], pl_prev))

        dot_half((g + 1) % 4, True)
        dot_half((g + 3) % 4, False)

        ownL = (g + 1) % 4
        ownR = (g - 1) % 4
        rowL = ownL * PB
        rowR = ownR * PB
        ownLz = (z + 1) % 4
        ownRz = (z - 1) % 4

        d1Lz[0].wait_recv()
        p2Ls[0] = (
            p_ref[pl.ds(rowL + z * ZB, ZB), :HN]
            + p1Lr[2, pl.ds(0, ZB), :].astype(F32)
        ).astype(BF)
        d2L = [rdma(p2Ls.at[0], p2Lr.at[0], s2Ls.at[0], s2Lr.at[0], z_next)]
        d1Rz[0].wait_recv()
        p2Rs[0] = (
            p_ref[pl.ds(rowR + z * ZB, ZB), HN:]
            + p1Rr[2, pl.ds(0, ZB), :].astype(F32)
        ).astype(BF)
        d2R = [rdma(p2Rs.at[0], p2Rr.at[0], s2Rs.at[0], s2Rr.at[0], z_prev)]

        for s in range(2):
            jL = (z - s - 1) % 4
            jR = (z + s + 1) % 4
            d2L[s].wait_recv()
            d1Lz[s + 1].wait_recv()
            p2Ls[s + 1] = (
                p_ref[pl.ds(rowL + jL * ZB, ZB), :HN]
                + p1Lr[2, pl.ds((s + 1) * ZB, ZB), :].astype(F32)
                + p2Lr[s].astype(F32)
            ).astype(BF)
            d2L.append(rdma(p2Ls.at[s + 1], p2Lr.at[s + 1],
                            s2Ls.at[s + 1], s2Lr.at[s + 1], z_next))
            d2R[s].wait_recv()
            d1Rz[s + 1].wait_recv()
            p2Rs[s + 1] = (
                p_ref[pl.ds(rowR + jR * ZB, ZB), HN:]
                + p1Rr[2, pl.ds((s + 1) * ZB, ZB), :].astype(F32)
                + p2Rr[s].astype(F32)
            ).astype(BF)
            d2R.append(rdma(p2Rs.at[s + 1], p2Rr.at[s + 1],
                            s2Rs.at[s + 1], s2Rr.at[s + 1], z_prev))

        def L3(j):
            r = rowL + ((z + 1 - j) % 4) * ZB
            return out_ref.at[pl.ds(r, ZB), pl.ds(0, HN)]

        def R3(j):
            r = rowR + ((z - 1 + j) % 4) * ZB
            return out_ref.at[pl.ds(r, ZB), pl.ds(HN, HN)]

        def L4(s, k):
            r = ((g + 1 - s) % 4) * PB + ((z + 1 - k) % 4) * ZB
            return out_ref.at[pl.ds(r, ZB), pl.ds(0, HN)]

        def R4(s, k):
            r = ((g - 1 + s) % 4) * PB + ((z - 1 + k) % 4) * ZB
            return out_ref.at[pl.ds(r, ZB), pl.ds(HN, HN)]

        d2L[2].wait_recv()
        d1Lz[3].wait_recv()
        vL = (p_ref[pl.ds(rowL + ownLz * ZB, ZB), :HN]
              + p1Lr[2, pl.ds(3 * ZB, ZB), :].astype(F32)
              + p2Lr[2].astype(F32))
        out_ref[pl.ds(rowL + ownLz * ZB, ZB), :HN] = vL.astype(BF)
        d3L = [rdma(L3(0), L3(0), s3Ls.at[0], s3Lr.at[0], z_next)]
        d4L = [rdma(L4(0, 0), L4(0, 0), s4Ls.at[0], s4Lr.at[0], pl_next)]
        d2R[2].wait_recv()
        d1Rz[3].wait_recv()
        vR = (p_ref[pl.ds(rowR + ownRz * ZB, ZB), HN:]
              + p1Rr[2, pl.ds(3 * ZB, ZB), :].astype(F32)
              + p2Rr[2].astype(F32))
        out_ref[pl.ds(rowR + ownRz * ZB, ZB), HN:] = vR.astype(BF)
        d3R = [rdma(R3(0), R3(0), s3Rs.at[0], s3Rr.at[0], z_prev)]
        d4R = [rdma(R4(0, 0), R4(0, 0), s4Rs.at[0], s4Rr.at[0], pl_prev)]

        for j in range(3):
            d3L[j].wait_recv()
            if j < 2:
                d3L.append(rdma(L3(j + 1), L3(j + 1),
                                s3Ls.at[j + 1], s3Lr.at[j + 1], z_next))
            d4L.append(rdma(L4(0, j + 1), L4(0, j + 1),
                            s4Ls.at[j + 1], s4Lr.at[j + 1], pl_next))
            d3R[j].wait_recv()
            if j < 2:
                d3R.append(rdma(R3(j + 1), R3(j + 1),
                                s3Rs.at[j + 1], s3Rr.at[j + 1], z_prev))
            d4R.append(rdma(R4(0, j + 1), R4(0, j + 1),
                            s4Rs.at[j + 1], s4Rr.at[j + 1], pl_prev))

        for s in range(3):
            for k in range(4):
                i = s * 4 + k
                d4L[i].wait_recv()
                if s < 2:
                    d4L.append(rdma(L4(s + 1, k), L4(s + 1, k),
                                    s4Ls.at[i + 4], s4Lr.at[i + 4], pl_next))
                d4R[i].wait_recv()
                if s < 2:
                    d4R.append(rdma(R4(s + 1, k), R4(s + 1, k),
                                    s4Rs.at[i + 4], s4Rr.at[i + 4], pl_prev))

        for d in pending:
            d.wait_send()

    return pl.pallas_call(
        body,
        out_shape=jax.ShapeDtypeStruct((M, N), BF),
        in_specs=[
            pl.BlockSpec(memory_space=pltpu.VMEM),
            pl.BlockSpec(memory_space=pltpu.VMEM),
        ],
        out_specs=pl.BlockSpec(memory_space=pltpu.VMEM),
        scratch_shapes=[
            pltpu.VMEM((M, N), F32),
            pltpu.VMEM((3, PB, HN), BF),
            pltpu.VMEM((3, PB, HN), BF),
            pltpu.VMEM((3, PB, HN), BF),
            pltpu.VMEM((3, PB, HN), BF),
            pltpu.VMEM((3, ZB, HN), BF),
            pltpu.VMEM((3, ZB, HN), BF),
            pltpu.VMEM((3, ZB, HN), BF),
            pltpu.VMEM((3, ZB, HN), BF),
            pltpu.SemaphoreType.DMA((12,)),
            pltpu.SemaphoreType.DMA((12,)),
            pltpu.SemaphoreType.DMA((12,)),
            pltpu.SemaphoreType.DMA((12,)),
            pltpu.SemaphoreType.DMA((3,)),
            pltpu.SemaphoreType.DMA((3,)),
            pltpu.SemaphoreType.DMA((3,)),
            pltpu.SemaphoreType.DMA((3,)),
            pltpu.SemaphoreType.DMA((3,)),
            pltpu.SemaphoreType.DMA((3,)),
            pltpu.SemaphoreType.DMA((3,)),
            pltpu.SemaphoreType.DMA((3,)),
            pltpu.SemaphoreType.DMA((12,)),
            pltpu.SemaphoreType.DMA((12,)),
            pltpu.SemaphoreType.DMA((12,)),
            pltpu.SemaphoreType.DMA((12,)),
        ],
        compiler_params=pltpu.CompilerParams(collective_id=0),
    )(A, B)


# device time: 63557 ns/iter; 2.6354x vs baseline; 1.1249x over previous
import jax
import jax.numpy as jnp
from jax import lax
from jax.experimental import pallas as pl
from jax.experimental.pallas import tpu as pltpu

N_DEV = 16
M = 1536
N = 1536
PB = M // 4
ZB = PB // 4
CL = 512
CZ = 256
BF = jnp.bfloat16
F32 = jnp.float32
NSTR = 4


def kernel(A, B):
    def body(a_ref, b_ref, out_ref, p_ref, *rest):
        bufs = rest[: NSTR * 4]
        sems = rest[NSTR * 4:]

        my = lax.axis_index("i")
        g = my % 4
        z = my // 4
        zbase = my - g

        pl_next = zbase + (g + 1) % 4
        pl_prev = zbase + (g - 1) % 4
        z_next = ((z + 1) % 4) * 4 + g
        z_prev = ((z - 1) % 4) * 4 + g

        barrier_sem = pltpu.get_barrier_semaphore()
        for nbr in (pl_next, pl_prev, z_next, z_prev):
            pl.semaphore_signal(
                barrier_sem, inc=1,
                device_id=(nbr,), device_id_type=pl.DeviceIdType.MESH,
            )
        pl.semaphore_wait(barrier_sem, 4)

        streams = []
        for si, (q1, q2, d1, d2, n1, n2, c0, cw) in enumerate([
            (g, z, 1, 1, pl_next, z_next, 0, CL),
            (g, z, -1, -1, pl_prev, z_prev, CL, CL),
            (z, g, 1, 1, z_next, pl_next, 2 * CL, CZ),
            (z, g, -1, -1, z_prev, pl_prev, 2 * CL + CZ, CZ),
        ]):
            p1s, p1r, p2s, p2r = bufs[si * 4: si * 4 + 4]
            s1s, s1r, s2s, s2r, s3s, s3r, s4s, s4r = sems[si * 8: si * 8 + 8]
            streams.append(dict(
                q1=q1, q2=q2, d1=d1, d2=d2, n1=n1, n2=n2, c0=c0, cw=cw,
                p1s=p1s, p1r=p1r, p2s=p2s, p2r=p2r,
                s1s=s1s, s1r=s1r, s2s=s2s, s2r=s2r,
                s3s=s3s, s3r=s3r, s4s=s4s, s4r=s4r,
                d1l=[], d1z=[], d2l=[], d3l=[], d4l=[],
            ))

        pending = []

        def rdma(src, dst, ssem, rsem, dev):
            d = pltpu.make_async_remote_copy(
                src_ref=src, dst_ref=dst, send_sem=ssem, recv_sem=rsem,
                device_id=(dev,), device_id_type=pl.DeviceIdType.MESH,
            )
            d.start()
            pending.append(d)
            return d

        def cols(S):
            return pl.ds(S["c0"], S["cw"])

        def dot(S, i):
            r = pl.ds(((i) % 4) * PB, PB)
            p_ref[r, cols(S)] = jnp.dot(
                a_ref[r, :].astype(BF),
                b_ref[:, cols(S)].astype(BF),
                preferred_element_type=F32,
            )

        def blk(S, i, j):
            return ((i) % 4) * PB + ((j) % 4) * ZB

        for S in streams:
            dot(S, S["q1"])
            S["p1s"][0] = p_ref[
                pl.ds((S["q1"] % 4) * PB, PB), cols(S)].astype(BF)
            S["d1l"].append(rdma(S["p1s"].at[0], S["p1r"].at[0],
                                 S["s1s"].at[0], S["s1r"].at[0], S["n1"]))
        for S in streams:
            dot(S, S["q1"] - S["d1"])
        for S in streams:
            j1 = (S["q1"] - S["d1"]) % 4
            S["d1l"][0].wait_recv()
            S["p1s"][1] = (
                p_ref[pl.ds(j1 * PB, PB), cols(S)]
                + S["p1r"][0].astype(F32)
            ).astype(BF)
            S["d1l"].append(rdma(S["p1s"].at[1], S["p1r"].at[1],
                                 S["s1s"].at[1], S["s1r"].at[1], S["n1"]))
        for S in streams:
            dot(S, S["q1"] - 2 * S["d1"])
        for S in streams:
            j2 = S["q1"] - 2 * S["d1"]
            S["d1l"][1].wait_recv()
            for k in range(4):
                zk = ((S["q2"] - S["d2"] * k) % 4) * ZB
                S["p1s"][2, pl.ds(k * ZB, ZB), :] = (
                    p_ref[pl.ds(blk(S, j2, 0) + zk, ZB), cols(S)]
                    + S["p1r"][1, pl.ds(zk, ZB), :].astype(F32)
                ).astype(BF)
                S["d1z"].append(
                    rdma(S["p1s"].at[2, pl.ds(k * ZB, ZB)],
                         S["p1r"].at[2, pl.ds(k * ZB, ZB)],
                         S["s1s"].at[2 + k], S["s1r"].at[2 + k], S["n1"]))
        for S in streams:
            dot(S, S["q1"] + S["d1"])

        for S in streams:
            own1 = S["q1"] + S["d1"]
            S["d1z"][0].wait_recv()
            S["p2s"][0] = (
                p_ref[pl.ds(blk(S, own1, S["q2"]), ZB), cols(S)]
                + S["p1r"][2, pl.ds(0, ZB), :].astype(F32)
            ).astype(BF)
            S["d2l"].append(rdma(S["p2s"].at[0], S["p2r"].at[0],
                                 S["s2s"].at[0], S["s2r"].at[0], S["n2"]))
        for s in range(2):
            for S in streams:
                own1 = S["q1"] + S["d1"]
                jj = S["q2"] - S["d2"] * (s + 1)
                S["d2l"][s].wait_recv()
                S["d1z"][s + 1].wait_recv()
                S["p2s"][s + 1] = (
                    p_ref[pl.ds(blk(S, own1, jj), ZB), cols(S)]
                    + S["p1r"][2, pl.ds((s + 1) * ZB, ZB), :].astype(F32)
                    + S["p2r"][s].astype(F32)
                ).astype(BF)
                S["d2l"].append(
                    rdma(S["p2s"].at[s + 1], S["p2r"].at[s + 1],
                         S["s2s"].at[s + 1], S["s2r"].at[s + 1], S["n2"]))

        def A3(S, j):
            r = blk(S, S["q1"] + S["d1"], S["q2"] + S["d2"] * (1 - j))
            return out_ref.at[pl.ds(r, ZB), cols(S)]

        def A4(S, s, k):
            r = blk(S, S["q1"] + S["d1"] * (1 - s),
                    S["q2"] + S["d2"] * (1 - k))
            return out_ref.at[pl.ds(r, ZB), cols(S)]

        for S in streams:
            own = blk(S, S["q1"] + S["d1"], S["q2"] + S["d2"])
            S["d2l"][2].wait_recv()
            S["d1z"][3].wait_recv()
            v = (p_ref[pl.ds(own, ZB), cols(S)]
                 + S["p1r"][2, pl.ds(3 * ZB, ZB), :].astype(F32)
                 + S["p2r"][2].astype(F32))
            out_ref[pl.ds(own, ZB), cols(S)] = v.astype(BF)
            S["d3l"].append(rdma(A3(S, 0), A3(S, 0),
                                 S["s3s"].at[0], S["s3r"].at[0], S["n2"]))
            S["d4l"].append(rdma(A4(S, 0, 0), A4(S, 0, 0),
                                 S["s4s"].at[0], S["s4r"].at[0], S["n1"]))

        for j in range(3):
            for S in streams:
                S["d3l"][j].wait_recv()
                if j < 2:
                    S["d3l"].append(
                        rdma(A3(S, j + 1), A3(S, j + 1),
                             S["s3s"].at[j + 1], S["s3r"].at[j + 1],
                             S["n2"]))
                S["d4l"].append(
                    rdma(A4(S, 0, j + 1), A4(S, 0, j + 1),
                         S["s4s"].at[j + 1], S["s4r"].at[j + 1], S["n1"]))

        for s in range(3):
            for k in range(4):
                i = s * 4 + k
                for S in streams:
                    S["d4l"][i].wait_recv()
                    if s < 2:
                        S["d4l"].append(
                            rdma(A4(S, s + 1, k), A4(S, s + 1, k),
                                 S["s4s"].at[i + 4], S["s4r"].at[i + 4],
                                 S["n1"]))

        for d in pending:
            d.wait_send()

    scratch = [pltpu.VMEM((M, N), F32)]
    for cw in (CL, CL, CZ, CZ):
        scratch += [
            pltpu.VMEM((3, PB, cw), BF),
            pltpu.VMEM((3, PB, cw), BF),
            pltpu.VMEM((3, ZB, cw), BF),
            pltpu.VMEM((3, ZB, cw), BF),
        ]
    for _ in range(NSTR):
        scratch += [
            pltpu.SemaphoreType.DMA((6,)),
            pltpu.SemaphoreType.DMA((6,)),
            pltpu.SemaphoreType.DMA((3,)),
            pltpu.SemaphoreType.DMA((3,)),
            pltpu.SemaphoreType.DMA((3,)),
            pltpu.SemaphoreType.DMA((3,)),
            pltpu.SemaphoreType.DMA((12,)),
            pltpu.SemaphoreType.DMA((12,)),
        ]

    return pl.pallas_call(
        body,
        out_shape=jax.ShapeDtypeStruct((M, N), BF),
        in_specs=[
            pl.BlockSpec(memory_space=pltpu.VMEM),
            pl.BlockSpec(memory_space=pltpu.VMEM),
        ],
        out_specs=pl.BlockSpec(memory_space=pltpu.VMEM),
        scratch_shapes=scratch,
        compiler_params=pltpu.CompilerParams(collective_id=0),
    )(A, B)
